# baseline (speedup 1.0000x reference)
import functools

import jax
import jax.numpy as jnp
from jax import lax
from jax.experimental import pallas as pl
from jax.experimental.pallas import tpu as pltpu
from jax.experimental.pallas import tpu_sc as plsc

EMBED_DIM = 64

_NC = 2
_NS = 16
_NW = _NC * _NS

_B = 4096
_T = 200
_TR = _T // 8
_BC = _B // 128
_UNITS = _T


@functools.lru_cache(maxsize=None)
def _make_kernel():
    mesh = plsc.VectorSubcoreMesh(core_axis_name="c", subcore_axis_name="s")

    @functools.partial(
        pl.kernel,
        mesh=mesh,
        compiler_params=pltpu.CompilerParams(
            use_tc_tiling_on_sc=False, needs_layout_passes=False),
        out_type=jax.ShapeDtypeStruct((_T, 8, _BC, 1024), jnp.float32),
        scratch_types=[
            pltpu.VMEM((_TR, 1024), jnp.int32),
            pltpu.VMEM((128, EMBED_DIM), jnp.float32),
            pltpu.VMEM((128, EMBED_DIM), jnp.float32),
            pltpu.VMEM((8, 1024), jnp.float32),
            pltpu.VMEM((8, 1024), jnp.float32),
            pltpu.SemaphoreType.DMA,
            pltpu.SemaphoreType.DMA,
            pltpu.SemaphoreType.DMA,
            pltpu.SemaphoreType.DMA,
        ],
    )
    def gather_kernel(table_hbm, idx_hbm, out_hbm, idx_v, rows_a, rows_b,
                      tb_a, tb_b, gsem_a, gsem_b, wsem_a, wsem_b):
        w = lax.axis_index("s") * _NC + lax.axis_index("c")
        rows = (rows_a, rows_b)
        tbs = (tb_a, tb_b)
        gsems = (gsem_a, gsem_b)
        wsems = (wsem_a, wsem_b)

        pltpu.sync_copy(idx_hbm.at[:, w, :], idx_v)

        def g_copy(k, p):
            tr, u = k // 8, k % 8
            return pltpu.make_async_copy(
                table_hbm.at[idx_v.at[tr, pl.ds(u * 128, 128)]],
                rows[p], gsems[p])

        def w_copy(k, p):
            return pltpu.make_async_copy(
                tbs[p], out_hbm.at[k, :, w, :], wsems[p])

        lanes = lax.iota(jnp.int32, 16)
        row_idx = [lanes + 16 * j for j in range(8)]

        def transpose(p):
            def per_dr(dr, carry):
                for u in range(8):
                    d = dr * 8 + u
                    col = jnp.full((16,), 0, jnp.int32) + d
                    for j in range(8):
                        vals = plsc.load_gather(rows[p], [row_idx[j], col])
                        tbs[p][dr, pl.ds(u * 128 + 16 * j, 16)] = vals
                return carry
            lax.fori_loop(0, 8, per_dr, 0)

        g_copy(0, 0).start()

        def body(k, carry):
            p = lax.rem(k, 2)

            def on(p):
                g_copy(k, p).wait()

                @pl.when(k < _UNITS - 1)
                def _():
                    g_copy(k + 1, 1 - p).start()

                @pl.when(k >= 2)
                def _():
                    w_copy(k - 2, p).wait()

                transpose(p)
                w_copy(k, p).start()

            @pl.when(p == 0)
            def _():
                on(0)

            @pl.when(p == 1)
            def _():
                on(1)

            return carry

        lax.fori_loop(0, _UNITS, body, 0)
        w_copy(_UNITS - 2, 0).wait()
        w_copy(_UNITS - 1, 1).wait()

    return gather_kernel


def kernel(indices, weight):
    b, t = indices.shape
    assert (b, t) == (_B, _T)
    idx_t = (indices.astype(jnp.int32).T
             .reshape(_TR, 8, _BC, 128)
             .transpose(0, 2, 1, 3)
             .reshape(_TR, _BC, 1024))
    w_flat = lax.optimization_barrier(weight.reshape(-1))
    w_lin = w_flat.reshape(weight.shape)
    out4 = _make_kernel()(w_lin, idx_t)
    out = (out4.reshape(_T, 8, _BC, 8, 128)
           .transpose(2, 4, 0, 1, 3)
           .reshape(_B, _T, EMBED_DIM))
    return out


# speedup vs baseline: 1.9788x; 1.9788x over previous
import functools

import jax
import jax.numpy as jnp
from jax import lax
from jax.experimental import pallas as pl
from jax.experimental.pallas import tpu as pltpu
from jax.experimental.pallas import tpu_sc as plsc

EMBED_DIM = 64

_NC = 2
_NS = 16
_NW = _NC * _NS

_B = 4096
_T = 200
_TR = _T // 8
_BC = _B // 128
_DEPTH = 4


@functools.lru_cache(maxsize=None)
def _make_kernel():
    mesh = plsc.VectorSubcoreMesh(core_axis_name="c", subcore_axis_name="s")

    @functools.partial(
        pl.kernel,
        mesh=mesh,
        compiler_params=pltpu.CompilerParams(
            use_tc_tiling_on_sc=False, needs_layout_passes=True),
        out_type=jax.ShapeDtypeStruct((_B, _T, 128), jnp.float32),
        scratch_types=[
            pltpu.VMEM((_TR, 1024), jnp.int32),
            pltpu.VMEM((128, EMBED_DIM), jnp.float32),
            pltpu.VMEM((128, EMBED_DIM), jnp.float32),
            pltpu.VMEM((128, EMBED_DIM), jnp.float32),
            pltpu.VMEM((128, EMBED_DIM), jnp.float32),
            pltpu.SemaphoreType.DMA,
            pltpu.SemaphoreType.DMA,
            pltpu.SemaphoreType.DMA,
            pltpu.SemaphoreType.DMA,
            pltpu.SemaphoreType.DMA,
            pltpu.SemaphoreType.DMA,
            pltpu.SemaphoreType.DMA,
            pltpu.SemaphoreType.DMA,
        ],
    )
    def gather_kernel(table_hbm, idx_hbm, out_hbm, idx_v, rows_0, rows_1,
                      rows_2, rows_3, gsem_0, gsem_1, gsem_2, gsem_3,
                      wsem_0, wsem_1, wsem_2, wsem_3):
        wk = lax.axis_index("s") * _NC + lax.axis_index("c")
        rows = (rows_0, rows_1, rows_2, rows_3)
        gsems = (gsem_0, gsem_1, gsem_2, gsem_3)
        wsems = (wsem_0, wsem_1, wsem_2, wsem_3)

        pltpu.sync_copy(idx_hbm.at[:, wk, :], idx_v)

        def g_copy(k, p):
            tr, ts = k // 8, k % 8
            return pltpu.make_async_copy(
                table_hbm.at[idx_v.at[tr, pl.ds(ts * 128, 128)]],
                rows[p], gsems[p])

        def w_copy(k, p):
            return pltpu.make_async_copy(
                rows[p],
                out_hbm.at[pl.ds(wk * 128, 128), k, pl.ds(0, EMBED_DIM)],
                wsems[p])

        for j in range(_DEPTH):
            g_copy(j, j).start()

        def body(r, carry):
            k0 = r * _DEPTH
            for j in range(_DEPTH):
                g_copy(k0 + j, j).wait()
                w_copy(k0 + j, j).start()

            @pl.when(r < _T // _DEPTH - 1)
            def _():
                for j in range(_DEPTH):
                    w_copy(k0 + j, j).wait()
                    g_copy(k0 + _DEPTH + j, j).start()

            return carry

        lax.fori_loop(0, _T // _DEPTH, body, 0)
        for j in range(_DEPTH):
            w_copy(_T - _DEPTH + j, j).wait()

    return gather_kernel


def kernel(indices, weight):
    b, t = indices.shape
    assert (b, t) == (_B, _T)
    idx_t = (indices.astype(jnp.int32).T
             .reshape(_TR, 8, _BC, 128)
             .transpose(0, 2, 1, 3)
             .reshape(_TR, _BC, 1024))
    out128 = _make_kernel()(weight, idx_t)
    return out128[:, :, :EMBED_DIM]


# speedup vs baseline: 2.8070x; 1.4185x over previous
import functools

import jax
import jax.numpy as jnp
from jax import lax
from jax.experimental import pallas as pl
from jax.experimental.pallas import tpu as pltpu
from jax.experimental.pallas import tpu_sc as plsc

EMBED_DIM = 64

_NC = 2
_NS = 16
_NW = _NC * _NS

_B = 4096
_T = 200
_TR = _T // 8
_BC = _B // 128
_DEPTH = 4

_VOCAB = 1000000
_TBLK = 8192


@functools.lru_cache(maxsize=None)
def _make_table_relayout():
    def body(x_ref, o_ref):
        t = x_ref[...].T
        o_ref[...] = jnp.concatenate([t, t], axis=1)

    grid = (_VOCAB + _TBLK - 1) // _TBLK
    return pl.pallas_call(
        body,
        grid=(grid,),
        in_specs=[pl.BlockSpec((EMBED_DIM, _TBLK), lambda i: (0, i))],
        out_specs=pl.BlockSpec((_TBLK, 128), lambda i: (i, 0)),
        out_shape=jax.ShapeDtypeStruct((_VOCAB, 128), jnp.float32),
    )


@functools.lru_cache(maxsize=None)
def _make_kernel():
    mesh = plsc.VectorSubcoreMesh(core_axis_name="c", subcore_axis_name="s")

    @functools.partial(
        pl.kernel,
        mesh=mesh,
        compiler_params=pltpu.CompilerParams(
            use_tc_tiling_on_sc=False, needs_layout_passes=True),
        out_type=jax.ShapeDtypeStruct((_B, _T, 128), jnp.float32),
        scratch_types=[
            pltpu.VMEM((_TR, 1024), jnp.int32),
            pltpu.VMEM((128, EMBED_DIM), jnp.float32),
            pltpu.VMEM((128, EMBED_DIM), jnp.float32),
            pltpu.VMEM((128, EMBED_DIM), jnp.float32),
            pltpu.VMEM((128, EMBED_DIM), jnp.float32),
            pltpu.SemaphoreType.DMA,
            pltpu.SemaphoreType.DMA,
            pltpu.SemaphoreType.DMA,
            pltpu.SemaphoreType.DMA,
            pltpu.SemaphoreType.DMA,
            pltpu.SemaphoreType.DMA,
            pltpu.SemaphoreType.DMA,
            pltpu.SemaphoreType.DMA,
        ],
    )
    def gather_kernel(table_hbm, idx_hbm, out_hbm, idx_v, rows_0, rows_1,
                      rows_2, rows_3, gsem_0, gsem_1, gsem_2, gsem_3,
                      wsem_0, wsem_1, wsem_2, wsem_3):
        wk = lax.axis_index("s") * _NC + lax.axis_index("c")
        rows = (rows_0, rows_1, rows_2, rows_3)
        gsems = (gsem_0, gsem_1, gsem_2, gsem_3)
        wsems = (wsem_0, wsem_1, wsem_2, wsem_3)

        pltpu.sync_copy(idx_hbm.at[:, wk, :], idx_v)

        def g_copy(k, p):
            tr, ts = k // 8, k % 8
            return pltpu.make_async_copy(
                table_hbm.at[idx_v.at[tr, pl.ds(ts * 128, 128)]],
                rows[p], gsems[p])

        def w_copy(k, p):
            return pltpu.make_async_copy(
                rows[p],
                out_hbm.at[pl.ds(wk * 128, 128), k, pl.ds(0, EMBED_DIM)],
                wsems[p])

        for j in range(_DEPTH):
            g_copy(j, j).start()

        def body(r, carry):
            k0 = r * _DEPTH
            for j in range(_DEPTH):
                g_copy(k0 + j, j).wait()
                w_copy(k0 + j, j).start()

            @pl.when(r < _T // _DEPTH - 1)
            def _():
                for j in range(_DEPTH):
                    w_copy(k0 + j, j).wait()
                    g_copy(k0 + _DEPTH + j, j).start()

            return carry

        lax.fori_loop(0, _T // _DEPTH, body, 0)
        for j in range(_DEPTH):
            w_copy(_T - _DEPTH + j, j).wait()

    return gather_kernel


def kernel(indices, weight):
    b, t = indices.shape
    assert (b, t) == (_B, _T)
    idx_t = ((indices.astype(jnp.int32) * 2).T
             .reshape(_TR, 8, _BC, 128)
             .transpose(0, 2, 1, 3)
             .reshape(_TR, _BC, 1024))
    w128 = _make_table_relayout()(weight.T)
    w2 = w128.reshape(2 * _VOCAB, EMBED_DIM)
    out128 = _make_kernel()(w2, idx_t)
    return out128[:, :, :EMBED_DIM]
